# baseline (speedup 1.0000x reference)
import functools

import jax
import jax.numpy as jnp
from jax import lax
from jax.experimental import pallas as pl
from jax.experimental.pallas import tpu as pltpu
from jax.experimental.pallas import tpu_sc as plsc

B = 16384
F = 32
R = 16
NC = 2
NS = 16
NW = NC * NS
BPW = B // NW
BPH = BPW // 2
LW = 128

BB = 2048
NB = B // BB


def _sc_gather_body(uid_hbm, iid_hbm, wu_hbm, wi_hbm, a_hbm, ub_hbm, ib_hbm,
                    ue4_out, wi4_out, a8_out, ub_out, ib_out,
                    uidx_v, iidx_v, urow_v, irow_v, arow_v,
                    ue4_v, wi4_v, a8_v, ub_v, ib_v, sem):
    wid = lax.axis_index("s") * NC + lax.axis_index("c")
    base = wid * BPW
    for h in range(2):
        hb = base + h * BPH
        pltpu.sync_copy(uid_hbm.at[pl.ds(hb, BPH)], uidx_v)
        pltpu.sync_copy(iid_hbm.at[pl.ds(hb, BPH)], iidx_v)
        for g in range(BPH // 16):
            s = pl.ds(g * 16, 16)
            u = uidx_v[s]
            i = iidx_v[s]
            urow_v[s] = lax.shift_right_logical(u, 2)
            irow_v[s] = lax.shift_right_logical(i, 2)
            arow_v[s] = lax.shift_right_logical(i, 3)
        c1 = pltpu.async_copy(wu_hbm.at[urow_v], ue4_v, sem)
        c2 = pltpu.async_copy(wi_hbm.at[irow_v], wi4_v, sem)
        c3 = pltpu.async_copy(a_hbm.at[arow_v], a8_v, sem)
        c4 = pltpu.async_copy(ub_hbm.at[uidx_v], ub_v, sem)
        c5 = pltpu.async_copy(ib_hbm.at[iidx_v], ib_v, sem)
        c1.wait(); c2.wait(); c3.wait(); c4.wait(); c5.wait()
        pltpu.sync_copy(ue4_v, ue4_out.at[pl.ds(hb, BPH)])
        pltpu.sync_copy(wi4_v, wi4_out.at[pl.ds(hb, BPH)])
        pltpu.sync_copy(a8_v, a8_out.at[pl.ds(hb, BPH)])
        pltpu.sync_copy(ub_v, ub_out.at[pl.ds(hb, BPH)])
        pltpu.sync_copy(ib_v, ib_out.at[pl.ds(hb, BPH)])


@functools.cache
def _make_sc_gather():
    return pl.kernel(
        _sc_gather_body,
        out_type=(
            jax.ShapeDtypeStruct((B, LW), jnp.float32),
            jax.ShapeDtypeStruct((B, LW), jnp.float32),
            jax.ShapeDtypeStruct((B, LW), jnp.float32),
            jax.ShapeDtypeStruct((B,), jnp.float32),
            jax.ShapeDtypeStruct((B,), jnp.float32),
        ),
        mesh=plsc.VectorSubcoreMesh(core_axis_name="c", subcore_axis_name="s",
                                    num_cores=NC, num_subcores=NS),
        compiler_params=pltpu.CompilerParams(use_tc_tiling_on_sc=True),
        scratch_types=[
            pltpu.VMEM((BPH,), jnp.int32),
            pltpu.VMEM((BPH,), jnp.int32),
            pltpu.VMEM((BPH,), jnp.int32),
            pltpu.VMEM((BPH,), jnp.int32),
            pltpu.VMEM((BPH,), jnp.int32),
            pltpu.VMEM((BPH, LW), jnp.float32),
            pltpu.VMEM((BPH, LW), jnp.float32),
            pltpu.VMEM((BPH, LW), jnp.float32),
            pltpu.VMEM((BPH,), jnp.float32),
            pltpu.VMEM((BPH,), jnp.float32),
            pltpu.SemaphoreType.DMA,
        ],
    )


def _tc_body(uf_ref, itf_ref, uid_ref, iid_ref, ue4_ref, wi4_ref, a8_ref,
             ub_ref, ib_ref,
             wuf_ref, wif_ref, bmat_ref, ufb_ref, ifb_ref, off_ref, out_ref):
    uf = uf_ref[...]
    itf = itf_ref[...]
    uid = uid_ref[...]
    iid = iid_ref[...]
    ue4 = ue4_ref[...]
    wi4 = wi4_ref[...]
    a8 = a8_ref[...]
    wuf = wuf_ref[...]
    wif = wif_ref[...]
    bmat = bmat_ref[...]

    um = uid & 3
    im3 = iid & 3
    im7 = iid & 7
    ue = jnp.zeros((ue4.shape[0], F), jnp.float32)
    wiv = jnp.zeros((ue4.shape[0], F), jnp.float32)
    for k in range(4):
        ue = ue + ue4[:, k * F:(k + 1) * F] * (um == k).astype(jnp.float32)
        wiv = wiv + wi4[:, k * F:(k + 1) * F] * (im3 == k).astype(jnp.float32)
    a = jnp.zeros((ue4.shape[0], R), jnp.float32)
    for k in range(8):
        a = a + a8[:, k * R:(k + 1) * R] * (im7 == k).astype(jnp.float32)

    ie = wiv + jnp.dot(a, bmat, preferred_element_type=jnp.float32)
    dsum = (jnp.dot(uf, wuf, preferred_element_type=jnp.float32)
            + jnp.dot(itf, wif, preferred_element_type=jnp.float32))
    s = ue + ie + dsum

    wuf2 = jnp.sum(wuf * wuf, axis=1)
    wif2 = jnp.sum(wif * wif, axis=1)
    sq = (jnp.sum(ue * ue, axis=1, keepdims=True)
          + jnp.sum(ie * ie, axis=1, keepdims=True)
          + jnp.sum(uf * uf * wuf2[None, :], axis=1, keepdims=True)
          + jnp.sum(itf * itf * wif2[None, :], axis=1, keepdims=True))
    quad = jnp.sum(s * s, axis=1, keepdims=True) - sq

    fb = (jnp.sum(uf * ufb_ref[...], axis=1, keepdims=True)
          + jnp.sum(itf * ifb_ref[...], axis=1, keepdims=True))
    out_ref[...] = (0.5 * quad + ub_ref[...] + ib_ref[...] + fb
                    + off_ref[0, 0])


def kernel(user_ids, item_ids, user_feats, item_feats, W_user, W_item,
           W_ufeat, W_ifeat, user_bias, item_bias, user_feat_bias,
           item_feat_bias, offset, A, Bmat):
    uid = user_ids.astype(jnp.int32)
    iid = item_ids.astype(jnp.int32)
    nu, f = W_user.shape
    ni = W_item.shape[0]
    r = A.shape[1]
    wu4 = W_user.reshape(nu * f // LW, LW)
    wi4t = W_item.reshape(ni * f // LW, LW)
    a8t = A.reshape(ni * r // LW, LW)
    ue4, wi4, a8, ub, ib = _make_sc_gather()(uid, iid, wu4, wi4t, a8t,
                                             user_bias, item_bias)

    nuf = user_feats.shape[1]
    nif = item_feats.shape[1]
    bspec = lambda shape: pl.BlockSpec(shape, lambda i: (i, 0))
    wspec = lambda shape: pl.BlockSpec(shape, lambda i: (0, 0))
    out = pl.pallas_call(
        _tc_body,
        grid=(NB,),
        in_specs=[
            bspec((BB, nuf)),
            bspec((BB, nif)),
            bspec((BB, 1)),
            bspec((BB, 1)),
            bspec((BB, LW)),
            bspec((BB, LW)),
            bspec((BB, LW)),
            bspec((BB, 1)),
            bspec((BB, 1)),
            wspec((nuf, F)),
            wspec((nif, F)),
            wspec((R, F)),
            wspec((1, nuf)),
            wspec((1, nif)),
            wspec((1, 1)),
        ],
        out_specs=bspec((BB, 1)),
        out_shape=jax.ShapeDtypeStruct((B, 1), jnp.float32),
    )(user_feats, item_feats, uid.reshape(B, 1), iid.reshape(B, 1),
      ue4, wi4, a8,
      ub.reshape(B, 1), ib.reshape(B, 1),
      W_ufeat, W_ifeat, Bmat,
      user_feat_bias.reshape(1, nuf), item_feat_bias.reshape(1, nif),
      offset.reshape(1, 1))
    return out.reshape(B)
